# baseline (speedup 1.0000x reference)
import functools

import numpy as np
import jax
import jax.numpy as jnp
from jax import lax
from jax.experimental import pallas as pl
from jax.experimental.pallas import tpu as pltpu
from jax.experimental.pallas import tpu_sc as plsc

_NUM_GROUP = 64
_GROUP_SIZE = 256


def _num_fps_points(num_points):
    np.random.seed(0)
    crop_rate = float(np.random.random())
    down_rate = _GROUP_SIZE / (num_points * crop_rate)
    return int(num_points * down_rate)


def _fps_kernel(x_ref, y_ref, z_ref,
                px_ref, py_ref, pz_ref, cx_ref, cy_ref, cz_ref,
                *, n, npts, ngroup, npad):
    X = x_ref[...]
    Y = y_ref[...]
    Z = z_ref[...]
    B = X.shape[0]
    I = jax.lax.broadcasted_iota(jnp.int32, (B, n), 1)
    Ip = jax.lax.broadcasted_iota(jnp.int32, (B, npad), 1)
    Ig = jax.lax.broadcasted_iota(jnp.int32, (B, ngroup), 1)

    def fps_step(i, dists, far, A, Aacc, Bc, Bacc, Cc, Cacc, Isrc, Iacc, nsent):
        oh = Isrc == far
        cx = jnp.sum(jnp.where(oh, A, 0.0), axis=1, keepdims=True)
        cy = jnp.sum(jnp.where(oh, Bc, 0.0), axis=1, keepdims=True)
        cz = jnp.sum(jnp.where(oh, Cc, 0.0), axis=1, keepdims=True)
        sel = Iacc == i
        Aacc = jnp.where(sel, cx, Aacc)
        Bacc = jnp.where(sel, cy, Bacc)
        Cacc = jnp.where(sel, cz, Cacc)
        dx = A - cx
        d = dx * dx
        dy = Bc - cy
        d = d + dy * dy
        dz = Cc - cz
        d = d + dz * dz
        dists = jnp.minimum(dists, d)
        m = jnp.max(dists, axis=1, keepdims=True)
        far = jnp.min(jnp.where(dists == m, Isrc, nsent), axis=1, keepdims=True)
        return dists, far, Aacc, Bacc, Cacc

    def body1(i, st):
        dists, far, PX, PY, PZ = st
        dists, far, PX, PY, PZ = fps_step(
            i, dists, far, X, PX, Y, PY, Z, PZ, I, Ip, n)
        return dists, far, PX, PY, PZ

    dists0 = jnp.full((B, n), 1e10, jnp.float32)
    far0 = jnp.zeros((B, 1), jnp.int32)
    P0 = jnp.zeros((B, npad), jnp.float32)
    _, _, PX, PY, PZ = jax.lax.fori_loop(
        0, npts, body1, (dists0, far0, P0, P0, P0))
    px_ref[...] = PX
    py_ref[...] = PY
    pz_ref[...] = PZ

    def body2(j, st):
        dists, far, CX, CY, CZ = st
        dists, far, CX, CY, CZ = fps_step(
            j, dists, far, PX, CX, PY, CY, PZ, CZ, Ip, Ig, npad)
        return dists, far, CX, CY, CZ

    dists20 = jnp.where(Ip < npts, jnp.float32(1e10), jnp.float32(-1.0))
    C0 = jnp.zeros((B, ngroup), jnp.float32)
    _, _, CX, CY, CZ = jax.lax.fori_loop(
        0, ngroup, body2, (dists20, far0, C0, C0, C0))
    cx_ref[...] = CX
    cy_ref[...] = CY
    cz_ref[...] = CZ


def _knn_sort_kernel(xt_ref, yt_ref, zt_ref, cx_ref, cy_ref, cz_ref,
                     io_ref, *, npts, npad, gsz):
    Xt = xt_ref[...]
    Yt = yt_ref[...]
    Zt = zt_ref[...]
    cx = cx_ref[...]
    cy = cy_ref[...]
    cz = cz_ref[...]
    bf = lambda a: a.astype(jnp.bfloat16).astype(jnp.float32)
    mm = (bf(cx) * bf(Xt) + bf(cy) * bf(Yt)) + bf(cz) * bf(Zt)
    c2 = (cx * cx + cy * cy) + cz * cz
    p2 = (Xt * Xt + Yt * Yt) + Zt * Zt
    key = -2.0 * mm
    key = key + c2
    key = key + p2

    row = jax.lax.broadcasted_iota(jnp.int32, key.shape, 0)
    key = jnp.where(row < npts, key, jnp.inf)
    idx = row
    logn = npad.bit_length() - 1
    for kk in range(1, logn + 1):
        asc = (row & (1 << kk)) == 0
        for jj in range(kk - 1, -1, -1):
            s = 1 << jj
            lo = (row & s) == 0
            kt = jnp.where(lo, jnp.roll(key, -s, axis=0),
                           jnp.roll(key, s, axis=0))
            it = jnp.where(lo, jnp.roll(idx, -s, axis=0),
                           jnp.roll(idx, s, axis=0))
            take_smaller = asc == lo
            theirs_smaller = (kt < key) | ((kt == key) & (it < idx))
            use_theirs = take_smaller == theirs_smaller
            key = jnp.where(use_theirs, kt, key)
            idx = jnp.where(use_theirs, it, idx)
    io_ref[...] = idx[:gsz]


def _make_sc_gather(B, npad, per):
    f32, i32 = jnp.float32, jnp.int32
    mesh = plsc.VectorSubcoreMesh(core_axis_name="c", subcore_axis_name="s")

    @functools.partial(
        pl.kernel, mesh=mesh,
        compiler_params=pltpu.CompilerParams(needs_layout_passes=False),
        out_type=[jax.ShapeDtypeStruct((B, per), f32)] * 3,
        scratch_types=[pltpu.VMEM((npad,), f32)] * 3
        + [pltpu.VMEM((per,), i32)]
        + [pltpu.VMEM((per,), f32)] * 6,
    )
    def sc_gather(px, py, pz, cxe, cye, cze, idx, ox, oy, oz,
                  tx, ty, tz, idxv, cxv, cyv, czv, bx, by, bz):
        wid = lax.axis_index("s") * 2 + lax.axis_index("c")
        pltpu.sync_copy(px.at[wid], tx)
        pltpu.sync_copy(py.at[wid], ty)
        pltpu.sync_copy(pz.at[wid], tz)
        pltpu.sync_copy(idx.at[wid], idxv)
        pltpu.sync_copy(cxe.at[wid], cxv)
        pltpu.sync_copy(cye.at[wid], cyv)
        pltpu.sync_copy(cze.at[wid], czv)

        def body(i, _):
            iv = idxv[pl.ds(i * 16, 16)]
            bx[pl.ds(i * 16, 16)] = (
                plsc.load_gather(tx, [iv]) - cxv[pl.ds(i * 16, 16)])
            by[pl.ds(i * 16, 16)] = (
                plsc.load_gather(ty, [iv]) - cyv[pl.ds(i * 16, 16)])
            bz[pl.ds(i * 16, 16)] = (
                plsc.load_gather(tz, [iv]) - czv[pl.ds(i * 16, 16)])
            return 0

        lax.fori_loop(0, per // 16, body, 0)
        pltpu.sync_copy(bx, ox.at[wid])
        pltpu.sync_copy(by, oy.at[wid])
        pltpu.sync_copy(bz, oz.at[wid])

    return sc_gather


def kernel(xyz, R_min, R_max):
    B, N, _ = xyz.shape
    npts = _num_fps_points(N)
    npad = max(128, 1 << (npts - 1).bit_length())
    f32 = jnp.float32
    xyz = xyz * (R_max - R_min) + R_min
    X = xyz[..., 0]
    Y = xyz[..., 1]
    Z = xyz[..., 2]

    fps = pl.pallas_call(
        functools.partial(_fps_kernel, n=N, npts=npts, ngroup=_NUM_GROUP,
                          npad=npad),
        out_shape=[jax.ShapeDtypeStruct((B, npad), f32)] * 3
        + [jax.ShapeDtypeStruct((B, _NUM_GROUP), f32)] * 3,
    )
    PX, PY, PZ, CX, CY, CZ = fps(X, Y, Z)

    R = B * _NUM_GROUP
    XRt = jnp.broadcast_to(PX.T[:, :, None], (npad, B, _NUM_GROUP)).reshape(npad, R)
    YRt = jnp.broadcast_to(PY.T[:, :, None], (npad, B, _NUM_GROUP)).reshape(npad, R)
    ZRt = jnp.broadcast_to(PZ.T[:, :, None], (npad, B, _NUM_GROUP)).reshape(npad, R)
    cxt = CX.reshape(1, R)
    cyt = CY.reshape(1, R)
    czt = CZ.reshape(1, R)

    knn_sort = pl.pallas_call(
        functools.partial(_knn_sort_kernel, npts=npts, npad=npad,
                          gsz=_GROUP_SIZE),
        out_shape=jax.ShapeDtypeStruct((_GROUP_SIZE, R), jnp.int32),
    )
    IO = knn_sort(XRt, YRt, ZRt, cxt, cyt, czt)
    per = _NUM_GROUP * _GROUP_SIZE
    IDXe = IO.T.reshape(B, per)
    CXE = jnp.broadcast_to(CX[:, :, None], (B, _NUM_GROUP, _GROUP_SIZE)).reshape(B, per)
    CYE = jnp.broadcast_to(CY[:, :, None], (B, _NUM_GROUP, _GROUP_SIZE)).reshape(B, per)
    CZE = jnp.broadcast_to(CZ[:, :, None], (B, _NUM_GROUP, _GROUP_SIZE)).reshape(B, per)

    sc_gather = _make_sc_gather(B, npad, per)
    NX, NY, NZ = sc_gather(PX, PY, PZ, CXE, CYE, CZE, IDXe)

    neighborhood = jnp.stack([NX, NY, NZ], axis=-1).reshape(
        B, _NUM_GROUP, _GROUP_SIZE, 3)
    center = jnp.stack([CX, CY, CZ], axis=-1).reshape(B, _NUM_GROUP, 3)
    return (neighborhood, center)


# speedup vs baseline: 1.0950x; 1.0950x over previous
import functools

import numpy as np
import jax
import jax.numpy as jnp
from jax import lax
from jax.experimental import pallas as pl
from jax.experimental.pallas import tpu as pltpu
from jax.experimental.pallas import tpu_sc as plsc

_NUM_GROUP = 64
_GROUP_SIZE = 256


def _num_fps_points(num_points):
    np.random.seed(0)
    crop_rate = float(np.random.random())
    down_rate = _GROUP_SIZE / (num_points * crop_rate)
    return int(num_points * down_rate)


def _fps_kernel(x_ref, y_ref, z_ref,
                px_ref, py_ref, pz_ref, cx_ref, cy_ref, cz_ref,
                *, n, npts, ngroup, npad):
    X = x_ref[...]
    Y = y_ref[...]
    Z = z_ref[...]
    B = X.shape[0]
    I = jax.lax.broadcasted_iota(jnp.int32, (B, n), 1)
    Ip = jax.lax.broadcasted_iota(jnp.int32, (B, npad), 1)
    Ig = jax.lax.broadcasted_iota(jnp.int32, (B, ngroup), 1)

    def fps_step(i, dists, far, A, Aacc, Bc, Bacc, Cc, Cacc, Isrc, Iacc, nsent):
        oh = Isrc == far
        cx = jnp.sum(jnp.where(oh, A, 0.0), axis=1, keepdims=True)
        cy = jnp.sum(jnp.where(oh, Bc, 0.0), axis=1, keepdims=True)
        cz = jnp.sum(jnp.where(oh, Cc, 0.0), axis=1, keepdims=True)
        sel = Iacc == i
        Aacc = jnp.where(sel, cx, Aacc)
        Bacc = jnp.where(sel, cy, Bacc)
        Cacc = jnp.where(sel, cz, Cacc)
        dx = A - cx
        d = dx * dx
        dy = Bc - cy
        d = d + dy * dy
        dz = Cc - cz
        d = d + dz * dz
        dists = jnp.minimum(dists, d)
        m = jnp.max(dists, axis=1, keepdims=True)
        far = jnp.min(jnp.where(dists == m, Isrc, nsent), axis=1, keepdims=True)
        return dists, far, Aacc, Bacc, Cacc

    unroll = 2 if npts % 2 == 0 else 1

    def body1(i, st):
        dists, far, PX, PY, PZ = st
        for u in range(unroll):
            dists, far, PX, PY, PZ = fps_step(
                i * unroll + u, dists, far, X, PX, Y, PY, Z, PZ, I, Ip, n)
        return dists, far, PX, PY, PZ

    dists0 = jnp.full((B, n), 1e10, jnp.float32)
    far0 = jnp.zeros((B, 1), jnp.int32)
    P0 = jnp.zeros((B, npad), jnp.float32)
    _, _, PX, PY, PZ = jax.lax.fori_loop(
        0, npts // unroll, body1, (dists0, far0, P0, P0, P0))
    px_ref[...] = PX
    py_ref[...] = PY
    pz_ref[...] = PZ

    def body2(j, st):
        dists, far, CX, CY, CZ = st
        dists, far, CX, CY, CZ = fps_step(
            j, dists, far, PX, CX, PY, CY, PZ, CZ, Ip, Ig, npad)
        return dists, far, CX, CY, CZ

    dists20 = jnp.where(Ip < npts, jnp.float32(1e10), jnp.float32(-1.0))
    C0 = jnp.zeros((B, ngroup), jnp.float32)
    _, _, CX, CY, CZ = jax.lax.fori_loop(
        0, ngroup, body2, (dists20, far0, C0, C0, C0))
    cx_ref[...] = CX
    cy_ref[...] = CY
    cz_ref[...] = CZ


def _knn_sort_kernel(xt_ref, yt_ref, zt_ref, cx_ref, cy_ref, cz_ref,
                     io_ref, *, npts, npad, gsz):
    Xt = xt_ref[...]
    Yt = yt_ref[...]
    Zt = zt_ref[...]
    cx = cx_ref[...]
    cy = cy_ref[...]
    cz = cz_ref[...]
    bf = lambda a: a.astype(jnp.bfloat16).astype(jnp.float32)
    mm = (bf(cx) * bf(Xt) + bf(cy) * bf(Yt)) + bf(cz) * bf(Zt)
    c2 = (cx * cx + cy * cy) + cz * cz
    p2 = (Xt * Xt + Yt * Yt) + Zt * Zt
    key = -2.0 * mm
    key = key + c2
    key = key + p2

    row = jax.lax.broadcasted_iota(jnp.int32, key.shape, 0)
    key = jnp.where(row < npts, key, jnp.inf)
    idx = row
    logn = npad.bit_length() - 1
    for kk in range(1, logn + 1):
        asc = (row & (1 << kk)) == 0
        for jj in range(kk - 1, -1, -1):
            s = 1 << jj
            lo = (row & s) == 0
            kt = jnp.where(lo, jnp.roll(key, -s, axis=0),
                           jnp.roll(key, s, axis=0))
            it = jnp.where(lo, jnp.roll(idx, -s, axis=0),
                           jnp.roll(idx, s, axis=0))
            take_smaller = asc == lo
            theirs_smaller = (kt < key) | ((kt == key) & (it < idx))
            use_theirs = take_smaller == theirs_smaller
            key = jnp.where(use_theirs, kt, key)
            idx = jnp.where(use_theirs, it, idx)
    io_ref[...] = idx[:gsz]


def _make_sc_gather(B, npad, per):
    f32, i32 = jnp.float32, jnp.int32
    mesh = plsc.VectorSubcoreMesh(core_axis_name="c", subcore_axis_name="s")

    @functools.partial(
        pl.kernel, mesh=mesh,
        compiler_params=pltpu.CompilerParams(needs_layout_passes=False),
        out_type=[jax.ShapeDtypeStruct((B, per), f32)] * 3,
        scratch_types=[pltpu.VMEM((npad,), f32)] * 3
        + [pltpu.VMEM((per,), i32)]
        + [pltpu.VMEM((per,), f32)] * 6,
    )
    def sc_gather(px, py, pz, cxe, cye, cze, idx, ox, oy, oz,
                  tx, ty, tz, idxv, cxv, cyv, czv, bx, by, bz):
        wid = lax.axis_index("s") * 2 + lax.axis_index("c")
        pltpu.sync_copy(px.at[wid], tx)
        pltpu.sync_copy(py.at[wid], ty)
        pltpu.sync_copy(pz.at[wid], tz)
        pltpu.sync_copy(idx.at[wid], idxv)
        pltpu.sync_copy(cxe.at[wid], cxv)
        pltpu.sync_copy(cye.at[wid], cyv)
        pltpu.sync_copy(cze.at[wid], czv)

        def body(i, _):
            iv = idxv[pl.ds(i * 16, 16)]
            bx[pl.ds(i * 16, 16)] = (
                plsc.load_gather(tx, [iv]) - cxv[pl.ds(i * 16, 16)])
            by[pl.ds(i * 16, 16)] = (
                plsc.load_gather(ty, [iv]) - cyv[pl.ds(i * 16, 16)])
            bz[pl.ds(i * 16, 16)] = (
                plsc.load_gather(tz, [iv]) - czv[pl.ds(i * 16, 16)])
            return 0

        lax.fori_loop(0, per // 16, body, 0)
        pltpu.sync_copy(bx, ox.at[wid])
        pltpu.sync_copy(by, oy.at[wid])
        pltpu.sync_copy(bz, oz.at[wid])

    return sc_gather


def kernel(xyz, R_min, R_max):
    B, N, _ = xyz.shape
    npts = _num_fps_points(N)
    npad = max(128, 1 << (npts - 1).bit_length())
    f32 = jnp.float32
    xyz = xyz * (R_max - R_min) + R_min
    X = xyz[..., 0]
    Y = xyz[..., 1]
    Z = xyz[..., 2]

    fps = pl.pallas_call(
        functools.partial(_fps_kernel, n=N, npts=npts, ngroup=_NUM_GROUP,
                          npad=npad),
        out_shape=[jax.ShapeDtypeStruct((B, npad), f32)] * 3
        + [jax.ShapeDtypeStruct((B, _NUM_GROUP), f32)] * 3,
    )
    PX, PY, PZ, CX, CY, CZ = fps(X, Y, Z)

    R = B * _NUM_GROUP
    XRt = jnp.broadcast_to(PX.T[:, :, None], (npad, B, _NUM_GROUP)).reshape(npad, R)
    YRt = jnp.broadcast_to(PY.T[:, :, None], (npad, B, _NUM_GROUP)).reshape(npad, R)
    ZRt = jnp.broadcast_to(PZ.T[:, :, None], (npad, B, _NUM_GROUP)).reshape(npad, R)
    cxt = CX.reshape(1, R)
    cyt = CY.reshape(1, R)
    czt = CZ.reshape(1, R)

    knn_sort = pl.pallas_call(
        functools.partial(_knn_sort_kernel, npts=npts, npad=npad,
                          gsz=_GROUP_SIZE),
        out_shape=jax.ShapeDtypeStruct((_GROUP_SIZE, R), jnp.int32),
    )
    IO = knn_sort(XRt, YRt, ZRt, cxt, cyt, czt)
    per = _NUM_GROUP * _GROUP_SIZE
    IDXe = IO.T.reshape(B, per)
    CXE = jnp.broadcast_to(CX[:, :, None], (B, _NUM_GROUP, _GROUP_SIZE)).reshape(B, per)
    CYE = jnp.broadcast_to(CY[:, :, None], (B, _NUM_GROUP, _GROUP_SIZE)).reshape(B, per)
    CZE = jnp.broadcast_to(CZ[:, :, None], (B, _NUM_GROUP, _GROUP_SIZE)).reshape(B, per)

    sc_gather = _make_sc_gather(B, npad, per)
    NX, NY, NZ = sc_gather(PX, PY, PZ, CXE, CYE, CZE, IDXe)

    neighborhood = jnp.stack([NX, NY, NZ], axis=-1).reshape(
        B, _NUM_GROUP, _GROUP_SIZE, 3)
    center = jnp.stack([CX, CY, CZ], axis=-1).reshape(B, _NUM_GROUP, 3)
    return (neighborhood, center)


# speedup vs baseline: 1.1395x; 1.0406x over previous
import functools

import numpy as np
import jax
import jax.numpy as jnp
from jax import lax
from jax.experimental import pallas as pl
from jax.experimental.pallas import tpu as pltpu
from jax.experimental.pallas import tpu_sc as plsc

_NUM_GROUP = 64
_GROUP_SIZE = 256


def _num_fps_points(num_points):
    np.random.seed(0)
    crop_rate = float(np.random.random())
    down_rate = _GROUP_SIZE / (num_points * crop_rate)
    return int(num_points * down_rate)


def _fps_kernel(x_ref, y_ref, z_ref,
                px_ref, py_ref, pz_ref, cx_ref, cy_ref, cz_ref,
                *, n, npts, ngroup, npad):
    X = x_ref[...]
    Y = y_ref[...]
    Z = z_ref[...]
    B = X.shape[0]
    I = jax.lax.broadcasted_iota(jnp.int32, (B, n), 1)
    Ip = jax.lax.broadcasted_iota(jnp.int32, (B, npad), 1)
    Ig = jax.lax.broadcasted_iota(jnp.int32, (B, ngroup), 1)

    def fps_step(i, dists, far, A, Aacc, Bc, Bacc, Cc, Cacc, Isrc, Iacc, nsent):
        oh = Isrc == far
        cx = jnp.sum(jnp.where(oh, A, 0.0), axis=1, keepdims=True)
        cy = jnp.sum(jnp.where(oh, Bc, 0.0), axis=1, keepdims=True)
        cz = jnp.sum(jnp.where(oh, Cc, 0.0), axis=1, keepdims=True)
        sel = Iacc == i
        Aacc = jnp.where(sel, cx, Aacc)
        Bacc = jnp.where(sel, cy, Bacc)
        Cacc = jnp.where(sel, cz, Cacc)
        dx = A - cx
        d = dx * dx
        dy = Bc - cy
        d = d + dy * dy
        dz = Cc - cz
        d = d + dz * dz
        dists = jnp.minimum(dists, d)
        m = jnp.max(dists, axis=1, keepdims=True)
        far = jnp.min(jnp.where(dists == m, Isrc, nsent), axis=1, keepdims=True)
        return dists, far, Aacc, Bacc, Cacc

    unroll = 8

    def body1(i, st):
        dists, far, PX, PY, PZ = st
        for u in range(unroll):
            dists, far, PX, PY, PZ = fps_step(
                i * unroll + u, dists, far, X, PX, Y, PY, Z, PZ, I, Ip, n)
        return dists, far, PX, PY, PZ

    dists0 = jnp.full((B, n), 1e10, jnp.float32)
    far0 = jnp.zeros((B, 1), jnp.int32)
    P0 = jnp.zeros((B, npad), jnp.float32)
    st = jax.lax.fori_loop(
        0, npts // unroll, body1, (dists0, far0, P0, P0, P0))
    dists, far, PX, PY, PZ = st
    for r in range((npts // unroll) * unroll, npts):
        dists, far, PX, PY, PZ = fps_step(
            r, dists, far, X, PX, Y, PY, Z, PZ, I, Ip, n)
    px_ref[...] = PX
    py_ref[...] = PY
    pz_ref[...] = PZ

    unroll2 = 8 if ngroup % 8 == 0 else 1

    def body2(j, st):
        dists, far, CX, CY, CZ = st
        for u in range(unroll2):
            dists, far, CX, CY, CZ = fps_step(
                j * unroll2 + u, dists, far, PX, CX, PY, CY, PZ, CZ, Ip, Ig,
                npad)
        return dists, far, CX, CY, CZ

    dists20 = jnp.where(Ip < npts, jnp.float32(1e10), jnp.float32(-1.0))
    C0 = jnp.zeros((B, ngroup), jnp.float32)
    _, _, CX, CY, CZ = jax.lax.fori_loop(
        0, ngroup // unroll2, body2, (dists20, far0, C0, C0, C0))
    cx_ref[...] = CX
    cy_ref[...] = CY
    cz_ref[...] = CZ


def _knn_sort_kernel(xt_ref, yt_ref, zt_ref, cx_ref, cy_ref, cz_ref,
                     io_ref, *, npts, npad, gsz):
    Xt = xt_ref[...]
    Yt = yt_ref[...]
    Zt = zt_ref[...]
    cx = cx_ref[...]
    cy = cy_ref[...]
    cz = cz_ref[...]
    bf = lambda a: a.astype(jnp.bfloat16).astype(jnp.float32)
    mm = (bf(cx) * bf(Xt) + bf(cy) * bf(Yt)) + bf(cz) * bf(Zt)
    c2 = (cx * cx + cy * cy) + cz * cz
    p2 = (Xt * Xt + Yt * Yt) + Zt * Zt
    key = -2.0 * mm
    key = key + c2
    key = key + p2

    row = jax.lax.broadcasted_iota(jnp.int32, key.shape, 0)
    key = jnp.where(row < npts, key, jnp.inf)
    idx = row
    logn = npad.bit_length() - 1
    for kk in range(1, logn + 1):
        asc = (row & (1 << kk)) == 0
        for jj in range(kk - 1, -1, -1):
            s = 1 << jj
            lo = (row & s) == 0
            kt = jnp.where(lo, jnp.roll(key, -s, axis=0),
                           jnp.roll(key, s, axis=0))
            it = jnp.where(lo, jnp.roll(idx, -s, axis=0),
                           jnp.roll(idx, s, axis=0))
            take_smaller = asc == lo
            theirs_smaller = (kt < key) | ((kt == key) & (it < idx))
            use_theirs = take_smaller == theirs_smaller
            key = jnp.where(use_theirs, kt, key)
            idx = jnp.where(use_theirs, it, idx)
    io_ref[...] = idx[:gsz]


def _make_sc_gather(B, npad, per):
    f32, i32 = jnp.float32, jnp.int32
    mesh = plsc.VectorSubcoreMesh(core_axis_name="c", subcore_axis_name="s")

    @functools.partial(
        pl.kernel, mesh=mesh,
        compiler_params=pltpu.CompilerParams(needs_layout_passes=False),
        out_type=[jax.ShapeDtypeStruct((B, per), f32)] * 3,
        scratch_types=[pltpu.VMEM((npad,), f32)] * 3
        + [pltpu.VMEM((per,), i32)]
        + [pltpu.VMEM((per,), f32)] * 6,
    )
    def sc_gather(px, py, pz, cxe, cye, cze, idx, ox, oy, oz,
                  tx, ty, tz, idxv, cxv, cyv, czv, bx, by, bz):
        wid = lax.axis_index("s") * 2 + lax.axis_index("c")
        pltpu.sync_copy(px.at[wid], tx)
        pltpu.sync_copy(py.at[wid], ty)
        pltpu.sync_copy(pz.at[wid], tz)
        pltpu.sync_copy(idx.at[wid], idxv)
        pltpu.sync_copy(cxe.at[wid], cxv)
        pltpu.sync_copy(cye.at[wid], cyv)
        pltpu.sync_copy(cze.at[wid], czv)

        def body(i, _):
            iv = idxv[pl.ds(i * 16, 16)]
            bx[pl.ds(i * 16, 16)] = (
                plsc.load_gather(tx, [iv]) - cxv[pl.ds(i * 16, 16)])
            by[pl.ds(i * 16, 16)] = (
                plsc.load_gather(ty, [iv]) - cyv[pl.ds(i * 16, 16)])
            bz[pl.ds(i * 16, 16)] = (
                plsc.load_gather(tz, [iv]) - czv[pl.ds(i * 16, 16)])
            return 0

        lax.fori_loop(0, per // 16, body, 0)
        pltpu.sync_copy(bx, ox.at[wid])
        pltpu.sync_copy(by, oy.at[wid])
        pltpu.sync_copy(bz, oz.at[wid])

    return sc_gather


def kernel(xyz, R_min, R_max):
    B, N, _ = xyz.shape
    npts = _num_fps_points(N)
    npad = max(128, 1 << (npts - 1).bit_length())
    f32 = jnp.float32
    xyz = xyz * (R_max - R_min) + R_min
    X = xyz[..., 0]
    Y = xyz[..., 1]
    Z = xyz[..., 2]

    fps = pl.pallas_call(
        functools.partial(_fps_kernel, n=N, npts=npts, ngroup=_NUM_GROUP,
                          npad=npad),
        out_shape=[jax.ShapeDtypeStruct((B, npad), f32)] * 3
        + [jax.ShapeDtypeStruct((B, _NUM_GROUP), f32)] * 3,
    )
    PX, PY, PZ, CX, CY, CZ = fps(X, Y, Z)

    R = B * _NUM_GROUP
    XRt = jnp.broadcast_to(PX.T[:, :, None], (npad, B, _NUM_GROUP)).reshape(npad, R)
    YRt = jnp.broadcast_to(PY.T[:, :, None], (npad, B, _NUM_GROUP)).reshape(npad, R)
    ZRt = jnp.broadcast_to(PZ.T[:, :, None], (npad, B, _NUM_GROUP)).reshape(npad, R)
    cxt = CX.reshape(1, R)
    cyt = CY.reshape(1, R)
    czt = CZ.reshape(1, R)

    knn_sort = pl.pallas_call(
        functools.partial(_knn_sort_kernel, npts=npts, npad=npad,
                          gsz=_GROUP_SIZE),
        out_shape=jax.ShapeDtypeStruct((_GROUP_SIZE, R), jnp.int32),
    )
    IO = knn_sort(XRt, YRt, ZRt, cxt, cyt, czt)
    per = _NUM_GROUP * _GROUP_SIZE
    IDXe = IO.T.reshape(B, per)
    CXE = jnp.broadcast_to(CX[:, :, None], (B, _NUM_GROUP, _GROUP_SIZE)).reshape(B, per)
    CYE = jnp.broadcast_to(CY[:, :, None], (B, _NUM_GROUP, _GROUP_SIZE)).reshape(B, per)
    CZE = jnp.broadcast_to(CZ[:, :, None], (B, _NUM_GROUP, _GROUP_SIZE)).reshape(B, per)

    sc_gather = _make_sc_gather(B, npad, per)
    NX, NY, NZ = sc_gather(PX, PY, PZ, CXE, CYE, CZE, IDXe)

    neighborhood = jnp.stack([NX, NY, NZ], axis=-1).reshape(
        B, _NUM_GROUP, _GROUP_SIZE, 3)
    center = jnp.stack([CX, CY, CZ], axis=-1).reshape(B, _NUM_GROUP, 3)
    return (neighborhood, center)


# speedup vs baseline: 1.1767x; 1.0326x over previous
import functools

import numpy as np
import jax
import jax.numpy as jnp
from jax import lax
from jax.experimental import pallas as pl
from jax.experimental.pallas import tpu as pltpu
from jax.experimental.pallas import tpu_sc as plsc

_NUM_GROUP = 64
_GROUP_SIZE = 256


def _num_fps_points(num_points):
    np.random.seed(0)
    crop_rate = float(np.random.random())
    down_rate = _GROUP_SIZE / (num_points * crop_rate)
    return int(num_points * down_rate)


def _fps_kernel(x_ref, y_ref, z_ref,
                px_ref, py_ref, pz_ref, cx_ref, cy_ref, cz_ref,
                *, n, npts, ngroup, npad):
    X = x_ref[...]
    Y = y_ref[...]
    Z = z_ref[...]
    B = X.shape[0]
    I = jax.lax.broadcasted_iota(jnp.int32, (B, n), 1)
    Ip = jax.lax.broadcasted_iota(jnp.int32, (B, npad), 1)
    Ig = jax.lax.broadcasted_iota(jnp.int32, (B, ngroup), 1)

    def fps_step(i, dists, far, A, Aacc, Bc, Bacc, Cc, Cacc, Isrc, Iacc, nsent):
        oh = Isrc == far
        cx = jnp.sum(jnp.where(oh, A, 0.0), axis=1, keepdims=True)
        cy = jnp.sum(jnp.where(oh, Bc, 0.0), axis=1, keepdims=True)
        cz = jnp.sum(jnp.where(oh, Cc, 0.0), axis=1, keepdims=True)
        sel = Iacc == i
        Aacc = jnp.where(sel, cx, Aacc)
        Bacc = jnp.where(sel, cy, Bacc)
        Cacc = jnp.where(sel, cz, Cacc)
        dx = A - cx
        d = dx * dx
        dy = Bc - cy
        d = d + dy * dy
        dz = Cc - cz
        d = d + dz * dz
        dists = jnp.minimum(dists, d)
        m = jnp.max(dists, axis=1, keepdims=True)
        far = jnp.min(jnp.where(dists == m, Isrc, nsent), axis=1, keepdims=True)
        return dists, far, Aacc, Bacc, Cacc

    unroll = 16

    def body1(i, st):
        dists, far, PX, PY, PZ = st
        for u in range(unroll):
            dists, far, PX, PY, PZ = fps_step(
                i * unroll + u, dists, far, X, PX, Y, PY, Z, PZ, I, Ip, n)
        return dists, far, PX, PY, PZ

    dists0 = jnp.full((B, n), 1e10, jnp.float32)
    far0 = jnp.zeros((B, 1), jnp.int32)
    P0 = jnp.zeros((B, npad), jnp.float32)
    st = jax.lax.fori_loop(
        0, npts // unroll, body1, (dists0, far0, P0, P0, P0))
    dists, far, PX, PY, PZ = st
    for r in range((npts // unroll) * unroll, npts):
        dists, far, PX, PY, PZ = fps_step(
            r, dists, far, X, PX, Y, PY, Z, PZ, I, Ip, n)
    px_ref[...] = PX
    py_ref[...] = PY
    pz_ref[...] = PZ

    unroll2 = 8 if ngroup % 8 == 0 else 1

    def body2(j, st):
        dists, far, CX, CY, CZ = st
        for u in range(unroll2):
            dists, far, CX, CY, CZ = fps_step(
                j * unroll2 + u, dists, far, PX, CX, PY, CY, PZ, CZ, Ip, Ig,
                npad)
        return dists, far, CX, CY, CZ

    dists20 = jnp.where(Ip < npts, jnp.float32(1e10), jnp.float32(-1.0))
    C0 = jnp.zeros((B, ngroup), jnp.float32)
    _, _, CX, CY, CZ = jax.lax.fori_loop(
        0, ngroup // unroll2, body2, (dists20, far0, C0, C0, C0))
    cx_ref[...] = CX
    cy_ref[...] = CY
    cz_ref[...] = CZ


def _knn_sort_kernel(xt_ref, yt_ref, zt_ref, cx_ref, cy_ref, cz_ref,
                     io_ref, *, npts, npad, gsz):
    Xt = xt_ref[...]
    Yt = yt_ref[...]
    Zt = zt_ref[...]
    cx = cx_ref[...]
    cy = cy_ref[...]
    cz = cz_ref[...]
    bf = lambda a: a.astype(jnp.bfloat16).astype(jnp.float32)
    mm = (bf(cx) * bf(Xt) + bf(cy) * bf(Yt)) + bf(cz) * bf(Zt)
    c2 = (cx * cx + cy * cy) + cz * cz
    p2 = (Xt * Xt + Yt * Yt) + Zt * Zt
    key = -2.0 * mm
    key = key + c2
    key = key + p2

    row = jax.lax.broadcasted_iota(jnp.int32, key.shape, 0)
    key = jnp.where(row < npts, key, jnp.inf)
    idx = row
    logn = npad.bit_length() - 1
    for kk in range(1, logn + 1):
        asc = (row & (1 << kk)) == 0
        for jj in range(kk - 1, -1, -1):
            s = 1 << jj
            lo = (row & s) == 0
            kt = jnp.where(lo, jnp.roll(key, -s, axis=0),
                           jnp.roll(key, s, axis=0))
            it = jnp.where(lo, jnp.roll(idx, -s, axis=0),
                           jnp.roll(idx, s, axis=0))
            take_smaller = asc == lo
            theirs_smaller = (kt < key) | ((kt == key) & (it < idx))
            use_theirs = take_smaller == theirs_smaller
            key = jnp.where(use_theirs, kt, key)
            idx = jnp.where(use_theirs, it, idx)
    io_ref[...] = idx[:gsz]


def _make_sc_gather(B, npad, per):
    f32, i32 = jnp.float32, jnp.int32
    mesh = plsc.VectorSubcoreMesh(core_axis_name="c", subcore_axis_name="s")

    @functools.partial(
        pl.kernel, mesh=mesh,
        compiler_params=pltpu.CompilerParams(needs_layout_passes=False),
        out_type=[jax.ShapeDtypeStruct((B, per), f32)] * 3,
        scratch_types=[pltpu.VMEM((npad,), f32)] * 3
        + [pltpu.VMEM((per,), i32)]
        + [pltpu.VMEM((per,), f32)] * 6,
    )
    def sc_gather(px, py, pz, cxe, cye, cze, idx, ox, oy, oz,
                  tx, ty, tz, idxv, cxv, cyv, czv, bx, by, bz):
        wid = lax.axis_index("s") * 2 + lax.axis_index("c")
        pltpu.sync_copy(px.at[wid], tx)
        pltpu.sync_copy(py.at[wid], ty)
        pltpu.sync_copy(pz.at[wid], tz)
        pltpu.sync_copy(idx.at[wid], idxv)
        pltpu.sync_copy(cxe.at[wid], cxv)
        pltpu.sync_copy(cye.at[wid], cyv)
        pltpu.sync_copy(cze.at[wid], czv)

        def body(i, _):
            iv = idxv[pl.ds(i * 16, 16)]
            bx[pl.ds(i * 16, 16)] = (
                plsc.load_gather(tx, [iv]) - cxv[pl.ds(i * 16, 16)])
            by[pl.ds(i * 16, 16)] = (
                plsc.load_gather(ty, [iv]) - cyv[pl.ds(i * 16, 16)])
            bz[pl.ds(i * 16, 16)] = (
                plsc.load_gather(tz, [iv]) - czv[pl.ds(i * 16, 16)])
            return 0

        lax.fori_loop(0, per // 16, body, 0)
        pltpu.sync_copy(bx, ox.at[wid])
        pltpu.sync_copy(by, oy.at[wid])
        pltpu.sync_copy(bz, oz.at[wid])

    return sc_gather


def kernel(xyz, R_min, R_max):
    B, N, _ = xyz.shape
    npts = _num_fps_points(N)
    npad = max(128, 1 << (npts - 1).bit_length())
    f32 = jnp.float32
    xyz = xyz * (R_max - R_min) + R_min
    X = xyz[..., 0]
    Y = xyz[..., 1]
    Z = xyz[..., 2]

    fps = pl.pallas_call(
        functools.partial(_fps_kernel, n=N, npts=npts, ngroup=_NUM_GROUP,
                          npad=npad),
        out_shape=[jax.ShapeDtypeStruct((B, npad), f32)] * 3
        + [jax.ShapeDtypeStruct((B, _NUM_GROUP), f32)] * 3,
    )
    PX, PY, PZ, CX, CY, CZ = fps(X, Y, Z)

    R = B * _NUM_GROUP
    XRt = jnp.broadcast_to(PX.T[:, :, None], (npad, B, _NUM_GROUP)).reshape(npad, R)
    YRt = jnp.broadcast_to(PY.T[:, :, None], (npad, B, _NUM_GROUP)).reshape(npad, R)
    ZRt = jnp.broadcast_to(PZ.T[:, :, None], (npad, B, _NUM_GROUP)).reshape(npad, R)
    cxt = CX.reshape(1, R)
    cyt = CY.reshape(1, R)
    czt = CZ.reshape(1, R)

    knn_sort = pl.pallas_call(
        functools.partial(_knn_sort_kernel, npts=npts, npad=npad,
                          gsz=_GROUP_SIZE),
        out_shape=jax.ShapeDtypeStruct((_GROUP_SIZE, R), jnp.int32),
    )
    IO = knn_sort(XRt, YRt, ZRt, cxt, cyt, czt)
    per = _NUM_GROUP * _GROUP_SIZE
    IDXe = IO.T.reshape(B, per)
    CXE = jnp.broadcast_to(CX[:, :, None], (B, _NUM_GROUP, _GROUP_SIZE)).reshape(B, per)
    CYE = jnp.broadcast_to(CY[:, :, None], (B, _NUM_GROUP, _GROUP_SIZE)).reshape(B, per)
    CZE = jnp.broadcast_to(CZ[:, :, None], (B, _NUM_GROUP, _GROUP_SIZE)).reshape(B, per)

    sc_gather = _make_sc_gather(B, npad, per)
    NX, NY, NZ = sc_gather(PX, PY, PZ, CXE, CYE, CZE, IDXe)

    neighborhood = jnp.stack([NX, NY, NZ], axis=-1).reshape(
        B, _NUM_GROUP, _GROUP_SIZE, 3)
    center = jnp.stack([CX, CY, CZ], axis=-1).reshape(B, _NUM_GROUP, 3)
    return (neighborhood, center)
